# baseline (speedup 1.0000x reference)
import functools

import jax
import jax.numpy as jnp
from jax import lax
from jax.experimental import pallas as pl
from jax.experimental.pallas import tpu as pltpu
from jax.experimental.pallas import tpu_sc as plsc

_PREC = jax.lax.Precision.HIGHEST

N = 4096
DEG = 32
E = N * DEG
K = 4
OBS = 33
HID = 64
OUT = 32
HEADS = 4
DH = OUT // HEADS

F32 = jnp.float32
I32 = jnp.int32


def _prep_body(ht, w1t, b1, w2t, b2, lng, lnb, wrt, wzt, wnt, br, bz, bn,
               hr, hz, hn, wsat, wsbt, wgt, ssrc, sdst, bs1r,
               ha_o, hb_o, xw_o, as_o, ad_o, mx_o):
    i = pl.program_id(0)
    x = jnp.maximum(jnp.dot(ht[...], w1t[...], preferred_element_type=F32) + b1[...], 0.0)
    x = jnp.maximum(jnp.dot(x, w2t[...], preferred_element_type=F32) + b2[...], 0.0)
    m = jnp.mean(x, axis=-1, keepdims=True)
    v = jnp.mean((x - m) ** 2, axis=-1, keepdims=True)
    e = (x - m) / jnp.sqrt(v + 1e-5) * lng[...] + lnb[...]
    ir = jnp.dot(e, wrt[...], preferred_element_type=F32) + br[...]
    iz = jnp.dot(e, wzt[...], preferred_element_type=F32) + bz[...]
    inn = jnp.dot(e, wnt[...], preferred_element_type=F32) + bn[...]
    r = jax.nn.sigmoid(ir + hr[...])
    z = jax.nn.sigmoid(iz + hz[...])
    nn = jnp.tanh(inn + r * hn[...])
    h = (1.0 - z) * nn
    zpad = jnp.zeros((h.shape[0], HID), F32)
    ha_o[...] = jnp.concatenate(
        [jnp.dot(h, wsat[...], preferred_element_type=F32) + bs1r[...],
         jnp.ones((h.shape[0], 1), F32), zpad[:, 1:]], axis=1)
    hb_o[...] = jnp.concatenate(
        [jnp.dot(h, wsbt[...], preferred_element_type=F32), zpad], axis=1)
    xw = jnp.dot(h, wgt[...], preferred_element_type=F32)
    xw_o[...] = xw
    a_s = jnp.dot(xw, ssrc[...], preferred_element_type=F32, precision=_PREC)
    a_d = jnp.dot(xw, sdst[...], preferred_element_type=F32, precision=_PREC)
    as_o[...] = a_s
    ad_o[...] = a_d
    ms = jnp.max(a_s, axis=0, keepdims=True)
    md = jnp.max(a_d, axis=0, keepdims=True)
    blockm = jnp.concatenate([ms, md, jnp.broadcast_to(ms, (6, HEADS))], axis=0)

    @pl.when(i == 0)
    def _():
        mx_o[...] = blockm

    @pl.when(i != 0)
    def _():
        mx_o[...] = jnp.maximum(mx_o[...], blockm)


def _prep_call(ht, w1t, b1, w2t, b2, lng, lnb, wrt, wzt, wnt, br, bz, bn,
               hr, hz, hn, wsat, wsbt, wgt, ssrc, sdst, bs1r):
    R = 512
    grid = (N // R,)
    row = lambda i: (i, 0)
    fix = lambda i: (0, 0)

    def full(a):
        return pl.BlockSpec(a.shape, fix)

    in_specs = [pl.BlockSpec((R, OBS), row)] + [
        full(a) for a in (w1t, b1, w2t, b2, lng, lnb, wrt, wzt, wnt, br, bz, bn,
                          hr, hz, hn, wsat, wsbt, wgt, ssrc, sdst, bs1r)]
    out_shape = [
        jax.ShapeDtypeStruct((N, 2 * HID), F32),
        jax.ShapeDtypeStruct((N, 2 * HID), F32),
        jax.ShapeDtypeStruct((N, OUT), F32),
        jax.ShapeDtypeStruct((N, HEADS), F32),
        jax.ShapeDtypeStruct((N, HEADS), F32),
        jax.ShapeDtypeStruct((8, HEADS), F32),
    ]
    out_specs = [
        pl.BlockSpec((R, 2 * HID), row),
        pl.BlockSpec((R, 2 * HID), row),
        pl.BlockSpec((R, OUT), row),
        pl.BlockSpec((R, HEADS), row),
        pl.BlockSpec((R, HEADS), row),
        pl.BlockSpec((8, HEADS), fix),
    ]
    return pl.pallas_call(_prep_body, grid=grid, in_specs=in_specs,
                          out_specs=out_specs, out_shape=out_shape)(
        ht, w1t, b1, w2t, b2, lng, lnb, wrt, wzt, wnt, br, bz, bn,
        hr, hz, hn, wsat, wsbt, wgt, ssrc, sdst, bs1r)


_NW = 32
_NPW = N // _NW
_CH = 32
_NCH = _NPW // _CH


_EPW = E // _NW
_BATCH = 512
_NBATCH = _EPW // _BATCH


def _gather_sc_body(hb_hbm, dst_hbm, g_hbm, dst_v, rows_v, sem):
    wid = lax.axis_index("s") * 2 + lax.axis_index("c")
    e0 = pl.multiple_of(wid * _EPW, _EPW)
    pltpu.sync_copy(dst_hbm.at[pl.ds(pl.multiple_of(e0 // 128, 8), _EPW // 128)],
                    dst_v)
    for b in range(_NBATCH):
        copies = [
            pltpu.async_copy(hb_hbm.at[dst_v.at[b * (_BATCH // 128) + g]],
                             rows_v.at[pl.ds(g * 128, 128)], sem)
            for g in range(_BATCH // 128)]
        for c in copies:
            c.wait()
        pltpu.sync_copy(rows_v, g_hbm.at[pl.ds(e0 + b * _BATCH, _BATCH)])


def _gather_sc_call(hb, dst2):
    mesh = plsc.VectorSubcoreMesh(core_axis_name="c", subcore_axis_name="s")
    kern = functools.partial(
        pl.kernel, mesh=mesh,
        out_type=jax.ShapeDtypeStruct((E, 2 * HID), F32),
        scratch_types=[
            pltpu.VMEM((_EPW // 128, 128), I32),
            pltpu.VMEM((_BATCH, 2 * HID), F32),
            pltpu.SemaphoreType.DMA,
        ])(_gather_sc_body)
    return kern(hb, dst2)


_ENB = 64
_EEB = _ENB * DEG


def _escore_body(g_ref, ha_ref, w2_ref, sc_o):
    rrow = lax.broadcasted_iota(I32, (_EEB, _ENB), 0) // DEG
    rcol = lax.broadcasted_iota(I32, (_EEB, _ENB), 1)
    rep = (rrow == rcol).astype(F32)
    ha_e = jnp.dot(rep, ha_ref[...], preferred_element_type=F32, precision=_PREC)
    u = jnp.maximum(g_ref[...] + ha_e, 0.0)
    u16 = u.astype(jnp.bfloat16).astype(F32)
    t = jnp.sum(u16 * w2_ref[...], axis=1, keepdims=True)
    sc_o[...] = jax.nn.sigmoid(t)


def _escore_call(g, ha, w2r):
    grid = (E // _EEB,)
    row = lambda i: (i, 0)
    fix = lambda i: (0, 0)
    return pl.pallas_call(
        _escore_body, grid=grid,
        in_specs=[pl.BlockSpec((_EEB, 2 * HID), row),
                  pl.BlockSpec((_ENB, 2 * HID), row),
                  pl.BlockSpec((1, 2 * HID), fix)],
        out_specs=pl.BlockSpec((_EEB, 1), row),
        out_shape=jax.ShapeDtypeStruct((E, 1), F32))(g, ha, w2r)


def _topk_body(s_ref, d_ref, w_o, d_o):
    s = s_ref[...]
    d = d_ref[...]
    R = s.shape[0]
    iota = lax.broadcasted_iota(I32, (R, DEG), 1)
    cur = s
    ws = []
    dsl = []
    for _ in range(K):
        m = jnp.max(cur, axis=1, keepdims=True)
        idx = jnp.min(jnp.where(cur == m, iota, DEG), axis=1, keepdims=True)
        sel = iota == idx
        dk = jnp.sum(jnp.where(sel, d, 0), axis=1, keepdims=True)
        ws.append(m)
        dsl.append(dk)
        cur = jnp.where(sel, -jnp.inf, cur)
    w_o[...] = jnp.concatenate(ws, axis=1)
    d_o[...] = jnp.concatenate(dsl, axis=1)


def _topk_call(scores, dmat):
    R = 512
    grid = (N // R,)
    row = lambda i: (i, 0)
    return pl.pallas_call(
        _topk_body, grid=grid,
        in_specs=[pl.BlockSpec((R, DEG), row), pl.BlockSpec((R, DEG), row)],
        out_specs=[pl.BlockSpec((R, K), row), pl.BlockSpec((R, K), row)],
        out_shape=[jax.ShapeDtypeStruct((N, K), F32),
                   jax.ShapeDtypeStruct((N, K), I32)])(scores, dmat)


_NB = 64
_EB = _NB * K


def _seg1_body(df, dft, as_ref, ad_ref, mx_ref, ex_o, den_o):
    i = pl.program_id(0)
    d = df[...]
    col = lax.broadcasted_iota(I32, (_EB, N), 1)
    oneh = (col == d).astype(F32)
    rowN = lax.broadcasted_iota(I32, (N, _EB), 0)
    onehT = (dft[...] == rowN).astype(F32)
    a_d_e = jnp.dot(oneh, ad_ref[...], preferred_element_type=F32, precision=_PREC)
    rrow = lax.broadcasted_iota(I32, (_EB, _NB), 0) // K
    rcol = lax.broadcasted_iota(I32, (_EB, _NB), 1)
    rep = (rrow == rcol).astype(F32)
    a_s_e = jnp.dot(rep, as_ref[...], preferred_element_type=F32, precision=_PREC)
    logit = a_s_e + a_d_e
    logit = jnp.where(logit >= 0, logit, 0.2 * logit)
    mx = mx_ref[...]
    big = mx[0:1, :] + mx[1:2, :]
    big = jnp.where(big >= 0, big, 0.2 * big)
    ex = jnp.exp(logit - big)
    ex_o[...] = ex

    @pl.when(i == 0)
    def _():
        den_o[...] = jnp.zeros_like(den_o)

    den_o[...] += jnp.dot(onehT, ex, preferred_element_type=F32, precision=_PREC)


def _seg1_call(df, dft, a_s, a_d, mx):
    grid = (N // _NB,)
    row = lambda i: (i, 0)
    fix = lambda i: (0, 0)
    return pl.pallas_call(
        _seg1_body, grid=grid,
        in_specs=[pl.BlockSpec((_EB, 1), row),
                  pl.BlockSpec((1, _EB), lambda i: (0, i)),
                  pl.BlockSpec((_NB, HEADS), row),
                  pl.BlockSpec((N, HEADS), fix),
                  pl.BlockSpec((8, HEADS), fix)],
        out_specs=[pl.BlockSpec((_EB, HEADS), row),
                   pl.BlockSpec((N, HEADS), fix)],
        out_shape=[jax.ShapeDtypeStruct((N * K, HEADS), F32),
                   jax.ShapeDtypeStruct((N, HEADS), F32)])(df, dft, a_s, a_d, mx)


def _seg2_body(df, dft, wf, ex_ref, den_ref, xw_ref, bias_ref, an_o, out_o):
    i = pl.program_id(0)
    nsteps = pl.num_programs(0)
    d = df[...]
    col = lax.broadcasted_iota(I32, (_EB, N), 1)
    oneh = (col == d).astype(F32)
    rowN = lax.broadcasted_iota(I32, (N, _EB), 0)
    onehT = (dft[...] == rowN).astype(F32)
    den_e = jnp.dot(oneh, den_ref[...], preferred_element_type=F32, precision=_PREC)
    alpha = ex_ref[...] / (den_e + 1e-16) * wf[...]
    rrow = lax.broadcasted_iota(I32, (_EB, _NB), 0) // K
    rcol = lax.broadcasted_iota(I32, (_EB, _NB), 1)
    rep = (rrow == rcol).astype(F32)
    repT = (lax.broadcasted_iota(I32, (_NB, _EB), 1) // K
            == lax.broadcasted_iota(I32, (_NB, _EB), 0)).astype(F32)
    rowsum = jnp.dot(repT, alpha, preferred_element_type=F32, precision=_PREC)
    rowsum = jnp.maximum(rowsum, 1e-9)
    an_o[...] = alpha / jnp.dot(rep, rowsum, preferred_element_type=F32, precision=_PREC)
    xw_e = jnp.dot(rep, xw_ref[...], preferred_element_type=F32, precision=_PREC)
    erow = lax.broadcasted_iota(I32, (HEADS, OUT), 0)
    ecol = lax.broadcasted_iota(I32, (HEADS, OUT), 1)
    expand = (ecol // DH == erow).astype(F32)
    msg = jnp.dot(alpha, expand, preferred_element_type=F32, precision=_PREC) * xw_e

    @pl.when(i == 0)
    def _():
        out_o[...] = jnp.zeros_like(out_o)

    out_o[...] += jnp.dot(onehT, msg, preferred_element_type=F32, precision=_PREC)

    @pl.when(i == nsteps - 1)
    def _():
        out_o[...] += bias_ref[...]


def _seg2_call(df, dft, wf, ex, den, xw, bias):
    grid = (N // _NB,)
    row = lambda i: (i, 0)
    fix = lambda i: (0, 0)
    return pl.pallas_call(
        _seg2_body, grid=grid,
        in_specs=[pl.BlockSpec((_EB, 1), row),
                  pl.BlockSpec((1, _EB), lambda i: (0, i)),
                  pl.BlockSpec((_EB, 1), row),
                  pl.BlockSpec((_EB, HEADS), row),
                  pl.BlockSpec((N, HEADS), fix),
                  pl.BlockSpec((_NB, OUT), row),
                  pl.BlockSpec((1, OUT), fix)],
        out_specs=[pl.BlockSpec((_EB, HEADS), row),
                   pl.BlockSpec((N, OUT), fix)],
        out_shape=[jax.ShapeDtypeStruct((N * K, HEADS), F32),
                   jax.ShapeDtypeStruct((N, OUT), F32)])(df, dft, wf, ex, den, xw, bias)


_AR = 256


def _abuild_body(d_ref, an_ref, a_o):
    d = d_ref[...]
    a = an_ref[0]
    col = lax.broadcasted_iota(I32, (_AR, N), 1)
    acc = jnp.zeros((_AR, N), F32)
    for k in range(K):
        acc += jnp.where(d[:, k:k + 1] == col, a[:, k:k + 1], 0.0)
    a_o[0, 0] = acc


def _abuild_call(dsel, an_t):
    grid = (HEADS, N // _AR)
    return pl.pallas_call(
        _abuild_body, grid=grid,
        in_specs=[pl.BlockSpec((_AR, K), lambda h, i: (i, 0)),
                  pl.BlockSpec((1, _AR, K), lambda h, i: (h, i, 0))],
        out_specs=pl.BlockSpec((1, 1, _AR, N), lambda h, i: (0, h, i, 0)),
        out_shape=jax.ShapeDtypeStruct((1, HEADS, N, N), F32))(dsel, an_t)


def kernel(H_t, src, dst, W1, b1, W2, b2, ln_g, ln_b, Wih, bih, Whh, bhh,
           Ws1, bs1, Ws2, bs2, Wg, att_src, att_dst, bias_g):
    ht = H_t.reshape(N, OBS)
    w1t = W1.T
    w2t = W2.T
    wrt = Wih[0:HID].T
    wzt = Wih[HID:2 * HID].T
    wnt = Wih[2 * HID:].T
    br = (bih[0:HID]).reshape(1, HID)
    bz = (bih[HID:2 * HID]).reshape(1, HID)
    bn = (bih[2 * HID:]).reshape(1, HID)
    hr = (bhh[0:HID]).reshape(1, HID)
    hz = (bhh[HID:2 * HID]).reshape(1, HID)
    hn = (bhh[2 * HID:]).reshape(1, HID)
    wsat = Ws1[:, :HID].T
    wsbt = Ws1[:, HID:].T
    wgt = Wg.T
    lane = jnp.arange(OUT)[:, None] // DH == jnp.arange(HEADS)[None, :]
    ssrc = att_src.reshape(OUT, 1) * lane.astype(F32)
    sdst = att_dst.reshape(OUT, 1) * lane.astype(F32)
    w2r = jnp.concatenate([Ws2.reshape(HID), bs2,
                           jnp.zeros((HID - 1,), F32)]).reshape(1, 2 * HID)
    w2r = w2r.astype(jnp.bfloat16).astype(F32)
    b1r = b1.reshape(1, -1)
    b2r = b2.reshape(1, -1)
    lngr = ln_g.reshape(1, -1)
    lnbr = ln_b.reshape(1, -1)
    bs1r = bs1.reshape(1, -1)
    biasr = bias_g.reshape(1, -1)

    ha, hb, xw, a_s, a_d, mx = _prep_call(
        ht, w1t, b1r, w2t, b2r, lngr, lnbr, wrt, wzt, wnt, br, bz, bn,
        hr, hz, hn, wsat, wsbt, wgt, ssrc, sdst, bs1r)

    g = _gather_sc_call(hb, dst.reshape(E // 128, 128))
    scores = _escore_call(g, ha, w2r)
    wsel, dsel = _topk_call(scores.reshape(N, DEG), dst.reshape(N, DEG))
    df = dsel.reshape(N * K, 1)
    dft = dsel.reshape(1, N * K)
    wf = wsel.reshape(N * K, 1)
    ex, den = _seg1_call(df, dft, a_s, a_d, mx)
    an, out_b = _seg2_call(df, dft, wf, ex, den, xw, biasr)
    an_t = an.reshape(N, K, HEADS).transpose(2, 0, 1)
    attn = _abuild_call(dsel, an_t)
    return out_b.reshape(1, N, OUT), attn


# speedup vs baseline: 2.4266x; 2.4266x over previous
import functools

import jax
import jax.numpy as jnp
from jax import lax
from jax.experimental import pallas as pl
from jax.experimental.pallas import tpu as pltpu
from jax.experimental.pallas import tpu_sc as plsc

_PREC = jax.lax.Precision.HIGHEST

N = 4096
DEG = 32
E = N * DEG
K = 4
OBS = 33
HID = 64
OUT = 32
HEADS = 4
DH = OUT // HEADS

F32 = jnp.float32
I32 = jnp.int32


def _rep_rows(x, times, rows):
    return jnp.broadcast_to(x[:, None, :], (x.shape[0], times, x.shape[1])
                            ).reshape(rows, x.shape[1])


def _gdot(sel, vals):
    hi = vals.astype(jnp.bfloat16).astype(F32)
    lo = vals - hi
    return (jnp.dot(sel, hi, preferred_element_type=F32)
            + jnp.dot(sel, lo, preferred_element_type=F32))


def _prep_body(ht, w1t, b1, w2t, b2, lng, lnb, wrt, wzt, wnt, br, bz, bn,
               hr, hz, hn, wsat, wsbt, wgt, ssrc, sdst, bs1r,
               ha_o, hb_o, xw_o, as_o, ad_o, mx_o):
    i = pl.program_id(0)
    x = jnp.maximum(jnp.dot(ht[...], w1t[...], preferred_element_type=F32) + b1[...], 0.0)
    x = jnp.maximum(jnp.dot(x, w2t[...], preferred_element_type=F32) + b2[...], 0.0)
    m = jnp.mean(x, axis=-1, keepdims=True)
    v = jnp.mean((x - m) ** 2, axis=-1, keepdims=True)
    e = (x - m) / jnp.sqrt(v + 1e-5) * lng[...] + lnb[...]
    ir = jnp.dot(e, wrt[...], preferred_element_type=F32) + br[...]
    iz = jnp.dot(e, wzt[...], preferred_element_type=F32) + bz[...]
    inn = jnp.dot(e, wnt[...], preferred_element_type=F32) + bn[...]
    r = jax.nn.sigmoid(ir + hr[...])
    z = jax.nn.sigmoid(iz + hz[...])
    nn = jnp.tanh(inn + r * hn[...])
    h = (1.0 - z) * nn
    zpad = jnp.zeros((h.shape[0], HID), F32)
    ha_o[...] = jnp.concatenate(
        [jnp.dot(h, wsat[...], preferred_element_type=F32) + bs1r[...],
         jnp.ones((h.shape[0], 1), F32), zpad[:, 1:]], axis=1)
    hb_o[...] = jnp.concatenate(
        [jnp.dot(h, wsbt[...], preferred_element_type=F32), zpad], axis=1)
    xw = jnp.dot(h, wgt[...], preferred_element_type=F32)
    xw_o[...] = xw
    a_s = jnp.dot(xw, ssrc[...], preferred_element_type=F32, precision=_PREC)
    a_d = jnp.dot(xw, sdst[...], preferred_element_type=F32, precision=_PREC)
    as_o[...] = a_s
    ad_o[...] = a_d
    ms = jnp.max(a_s, axis=0, keepdims=True)
    md = jnp.max(a_d, axis=0, keepdims=True)
    blockm = jnp.concatenate([ms, md, jnp.broadcast_to(ms, (6, HEADS))], axis=0)

    @pl.when(i == 0)
    def _():
        mx_o[...] = blockm

    @pl.when(i != 0)
    def _():
        mx_o[...] = jnp.maximum(mx_o[...], blockm)


def _prep_call(ht, w1t, b1, w2t, b2, lng, lnb, wrt, wzt, wnt, br, bz, bn,
               hr, hz, hn, wsat, wsbt, wgt, ssrc, sdst, bs1r):
    R = 512
    grid = (N // R,)
    row = lambda i: (i, 0)
    fix = lambda i: (0, 0)

    def full(a):
        return pl.BlockSpec(a.shape, fix)

    in_specs = [pl.BlockSpec((R, OBS), row)] + [
        full(a) for a in (w1t, b1, w2t, b2, lng, lnb, wrt, wzt, wnt, br, bz, bn,
                          hr, hz, hn, wsat, wsbt, wgt, ssrc, sdst, bs1r)]
    out_shape = [
        jax.ShapeDtypeStruct((N, 2 * HID), F32),
        jax.ShapeDtypeStruct((N, 2 * HID), F32),
        jax.ShapeDtypeStruct((N, OUT), F32),
        jax.ShapeDtypeStruct((N, HEADS), F32),
        jax.ShapeDtypeStruct((N, HEADS), F32),
        jax.ShapeDtypeStruct((8, HEADS), F32),
    ]
    out_specs = [
        pl.BlockSpec((R, 2 * HID), row),
        pl.BlockSpec((R, 2 * HID), row),
        pl.BlockSpec((R, OUT), row),
        pl.BlockSpec((R, HEADS), row),
        pl.BlockSpec((R, HEADS), row),
        pl.BlockSpec((8, HEADS), fix),
    ]
    return pl.pallas_call(_prep_body, grid=grid, in_specs=in_specs,
                          out_specs=out_specs, out_shape=out_shape)(
        ht, w1t, b1, w2t, b2, lng, lnb, wrt, wzt, wnt, br, bz, bn,
        hr, hz, hn, wsat, wsbt, wgt, ssrc, sdst, bs1r)


_NW = 32
_NPW = N // _NW
_CH = 32
_NCH = _NPW // _CH


_EPW = E // _NW
_BATCH = 512
_NBATCH = _EPW // _BATCH


def _gather_sc_body(hb_hbm, dst_hbm, g_hbm, dst_v, rows_v, sem):
    wid = lax.axis_index("s") * 2 + lax.axis_index("c")
    e0 = pl.multiple_of(wid * _EPW, _EPW)
    pltpu.sync_copy(dst_hbm.at[pl.ds(pl.multiple_of(e0 // 128, 8), _EPW // 128)],
                    dst_v)
    for b in range(_NBATCH):
        copies = [
            pltpu.async_copy(hb_hbm.at[dst_v.at[b * (_BATCH // 128) + g]],
                             rows_v.at[pl.ds(g * 128, 128)], sem)
            for g in range(_BATCH // 128)]
        for c in copies:
            c.wait()
        pltpu.sync_copy(rows_v, g_hbm.at[pl.ds(e0 + b * _BATCH, _BATCH)])


def _gather_sc_call(hb, dst2):
    mesh = plsc.VectorSubcoreMesh(core_axis_name="c", subcore_axis_name="s")
    kern = functools.partial(
        pl.kernel, mesh=mesh,
        out_type=jax.ShapeDtypeStruct((E, 2 * HID), F32),
        scratch_types=[
            pltpu.VMEM((_EPW // 128, 128), I32),
            pltpu.VMEM((_BATCH, 2 * HID), F32),
            pltpu.SemaphoreType.DMA,
        ])(_gather_sc_body)
    return kern(hb, dst2)


_ENB = 64
_EEB = _ENB * DEG


def _escore_body(g_ref, ha_ref, w2_ref, sc_o):
    ha_e = _rep_rows(ha_ref[...], DEG, _EEB)
    u = jnp.maximum(g_ref[...] + ha_e, 0.0)
    u16 = u.astype(jnp.bfloat16).astype(F32)
    t = jnp.sum(u16 * w2_ref[...], axis=1, keepdims=True)
    sc_o[...] = jax.nn.sigmoid(t)


def _escore_call(g, ha, w2r):
    grid = (E // _EEB,)
    row = lambda i: (i, 0)
    fix = lambda i: (0, 0)
    return pl.pallas_call(
        _escore_body, grid=grid,
        in_specs=[pl.BlockSpec((_EEB, 2 * HID), row),
                  pl.BlockSpec((_ENB, 2 * HID), row),
                  pl.BlockSpec((1, 2 * HID), fix)],
        out_specs=pl.BlockSpec((_EEB, 1), row),
        out_shape=jax.ShapeDtypeStruct((E, 1), F32))(g, ha, w2r)


def _topk_body(s_ref, d_ref, w_o, d_o):
    s = s_ref[...]
    d = d_ref[...]
    R = s.shape[0]
    iota = lax.broadcasted_iota(I32, (R, DEG), 1)
    cur = s
    ws = []
    dsl = []
    for _ in range(K):
        m = jnp.max(cur, axis=1, keepdims=True)
        idx = jnp.min(jnp.where(cur == m, iota, DEG), axis=1, keepdims=True)
        sel = iota == idx
        dk = jnp.sum(jnp.where(sel, d, 0), axis=1, keepdims=True)
        ws.append(m)
        dsl.append(dk)
        cur = jnp.where(sel, -jnp.inf, cur)
    w_o[...] = jnp.concatenate(ws, axis=1)
    d_o[...] = jnp.concatenate(dsl, axis=1)


def _topk_call(scores, dmat):
    R = 512
    grid = (N // R,)
    row = lambda i: (i, 0)
    return pl.pallas_call(
        _topk_body, grid=grid,
        in_specs=[pl.BlockSpec((R, DEG), row), pl.BlockSpec((R, DEG), row)],
        out_specs=[pl.BlockSpec((R, K), row), pl.BlockSpec((R, K), row)],
        out_shape=[jax.ShapeDtypeStruct((N, K), F32),
                   jax.ShapeDtypeStruct((N, K), I32)])(scores, dmat)


_NB = 64
_EB = _NB * K


def _seg1_body(df, dft, as_ref, ad_ref, mx_ref, ex_o, den_o):
    i = pl.program_id(0)
    d = df[...]
    col = lax.broadcasted_iota(I32, (_EB, N), 1)
    oneh = (col == d).astype(F32)
    rowN = lax.broadcasted_iota(I32, (N, _EB), 0)
    onehT = (dft[...] == rowN).astype(F32)
    a_d_e = _gdot(oneh, ad_ref[...])
    a_s_e = _rep_rows(as_ref[...], K, _EB)
    logit = a_s_e + a_d_e
    logit = jnp.where(logit >= 0, logit, 0.2 * logit)
    mx = mx_ref[...]
    big = mx[0:1, :] + mx[1:2, :]
    big = jnp.where(big >= 0, big, 0.2 * big)
    ex = jnp.exp(logit - big)
    ex_o[...] = ex

    @pl.when(i == 0)
    def _():
        den_o[...] = jnp.zeros_like(den_o)

    den_o[...] += _gdot(onehT, ex)


def _seg1_call(df, dft, a_s, a_d, mx):
    grid = (N // _NB,)
    row = lambda i: (i, 0)
    fix = lambda i: (0, 0)
    return pl.pallas_call(
        _seg1_body, grid=grid,
        in_specs=[pl.BlockSpec((_EB, 1), row),
                  pl.BlockSpec((1, _EB), lambda i: (0, i)),
                  pl.BlockSpec((_NB, HEADS), row),
                  pl.BlockSpec((N, HEADS), fix),
                  pl.BlockSpec((8, HEADS), fix)],
        out_specs=[pl.BlockSpec((_EB, HEADS), row),
                   pl.BlockSpec((N, HEADS), fix)],
        out_shape=[jax.ShapeDtypeStruct((N * K, HEADS), F32),
                   jax.ShapeDtypeStruct((N, HEADS), F32)])(df, dft, a_s, a_d, mx)


def _seg2_body(df, dft, wf, ex_ref, den_ref, xw_ref, bias_ref, an_o, out_o):
    i = pl.program_id(0)
    nsteps = pl.num_programs(0)
    d = df[...]
    col = lax.broadcasted_iota(I32, (_EB, N), 1)
    oneh = (col == d).astype(F32)
    rowN = lax.broadcasted_iota(I32, (N, _EB), 0)
    onehT = (dft[...] == rowN).astype(F32)
    den_e = _gdot(oneh, den_ref[...])
    alpha = ex_ref[...] / (den_e + 1e-16) * wf[...]
    repT = (lax.broadcasted_iota(I32, (_NB, _EB), 1) // K
            == lax.broadcasted_iota(I32, (_NB, _EB), 0)).astype(F32)
    rowsum = jnp.dot(repT, alpha, preferred_element_type=F32, precision=_PREC)
    rowsum = jnp.maximum(rowsum, 1e-9)
    an_o[...] = alpha / _rep_rows(rowsum, K, _EB)
    xw_e = _rep_rows(xw_ref[...], K, _EB)
    erow = lax.broadcasted_iota(I32, (HEADS, OUT), 0)
    ecol = lax.broadcasted_iota(I32, (HEADS, OUT), 1)
    expand = (ecol // DH == erow).astype(F32)
    msg = jnp.dot(alpha, expand, preferred_element_type=F32, precision=_PREC) * xw_e

    @pl.when(i == 0)
    def _():
        out_o[...] = jnp.zeros_like(out_o)

    out_o[...] += _gdot(onehT, msg)

    @pl.when(i == nsteps - 1)
    def _():
        out_o[...] += bias_ref[...]


def _seg2_call(df, dft, wf, ex, den, xw, bias):
    grid = (N // _NB,)
    row = lambda i: (i, 0)
    fix = lambda i: (0, 0)
    return pl.pallas_call(
        _seg2_body, grid=grid,
        in_specs=[pl.BlockSpec((_EB, 1), row),
                  pl.BlockSpec((1, _EB), lambda i: (0, i)),
                  pl.BlockSpec((_EB, 1), row),
                  pl.BlockSpec((_EB, HEADS), row),
                  pl.BlockSpec((N, HEADS), fix),
                  pl.BlockSpec((_NB, OUT), row),
                  pl.BlockSpec((1, OUT), fix)],
        out_specs=[pl.BlockSpec((_EB, HEADS), row),
                   pl.BlockSpec((N, OUT), fix)],
        out_shape=[jax.ShapeDtypeStruct((N * K, HEADS), F32),
                   jax.ShapeDtypeStruct((N, OUT), F32)])(df, dft, wf, ex, den, xw, bias)


_AR = 256


def _abuild_body(d_ref, an_ref, a_o):
    d = d_ref[...]
    a = an_ref[0]
    col = lax.broadcasted_iota(I32, (_AR, N), 1)
    acc = jnp.zeros((_AR, N), F32)
    for k in range(K):
        acc += jnp.where(d[:, k:k + 1] == col, a[:, k:k + 1], 0.0)
    a_o[0, 0] = acc


def _abuild_call(dsel, an_t):
    grid = (HEADS, N // _AR)
    return pl.pallas_call(
        _abuild_body, grid=grid,
        in_specs=[pl.BlockSpec((_AR, K), lambda h, i: (i, 0)),
                  pl.BlockSpec((1, _AR, K), lambda h, i: (h, i, 0))],
        out_specs=pl.BlockSpec((1, 1, _AR, N), lambda h, i: (0, h, i, 0)),
        out_shape=jax.ShapeDtypeStruct((1, HEADS, N, N), F32))(dsel, an_t)


def kernel(H_t, src, dst, W1, b1, W2, b2, ln_g, ln_b, Wih, bih, Whh, bhh,
           Ws1, bs1, Ws2, bs2, Wg, att_src, att_dst, bias_g):
    ht = H_t.reshape(N, OBS)
    w1t = W1.T
    w2t = W2.T
    wrt = Wih[0:HID].T
    wzt = Wih[HID:2 * HID].T
    wnt = Wih[2 * HID:].T
    br = (bih[0:HID]).reshape(1, HID)
    bz = (bih[HID:2 * HID]).reshape(1, HID)
    bn = (bih[2 * HID:]).reshape(1, HID)
    hr = (bhh[0:HID]).reshape(1, HID)
    hz = (bhh[HID:2 * HID]).reshape(1, HID)
    hn = (bhh[2 * HID:]).reshape(1, HID)
    wsat = Ws1[:, :HID].T
    wsbt = Ws1[:, HID:].T
    wgt = Wg.T
    lane = jnp.arange(OUT)[:, None] // DH == jnp.arange(HEADS)[None, :]
    ssrc = att_src.reshape(OUT, 1) * lane.astype(F32)
    sdst = att_dst.reshape(OUT, 1) * lane.astype(F32)
    w2r = jnp.concatenate([Ws2.reshape(HID), bs2,
                           jnp.zeros((HID - 1,), F32)]).reshape(1, 2 * HID)
    w2r = w2r.astype(jnp.bfloat16).astype(F32)
    b1r = b1.reshape(1, -1)
    b2r = b2.reshape(1, -1)
    lngr = ln_g.reshape(1, -1)
    lnbr = ln_b.reshape(1, -1)
    bs1r = bs1.reshape(1, -1)
    biasr = bias_g.reshape(1, -1)

    ha, hb, xw, a_s, a_d, mx = _prep_call(
        ht, w1t, b1r, w2t, b2r, lngr, lnbr, wrt, wzt, wnt, br, bz, bn,
        hr, hz, hn, wsat, wsbt, wgt, ssrc, sdst, bs1r)

    g = _gather_sc_call(hb, dst.reshape(E // 128, 128))
    scores = _escore_call(g, ha, w2r)
    wsel, dsel = _topk_call(scores.reshape(N, DEG), dst.reshape(N, DEG))
    df = dsel.reshape(N * K, 1)
    dft = dsel.reshape(1, N * K)
    wf = wsel.reshape(N * K, 1)
    ex, den = _seg1_call(df, dft, a_s, a_d, mx)
    an, out_b = _seg2_call(df, dft, wf, ex, den, xw, biasr)
    an_t = an.reshape(N, K, HEADS).transpose(2, 0, 1)
    attn = _abuild_call(dsel, an_t)
    return out_b.reshape(1, N, OUT), attn
